# baseline (speedup 1.0000x reference)
import functools

import jax
import jax.numpy as jnp
from jax import lax
from jax.experimental import pallas as pl
from jax.experimental.pallas import tpu as pltpu
from jax.experimental.pallas import tpu_sc as plsc

NC = 2
NSUB = 16
NW = NC * NSUB
L = 16

B = 16384
D = 64
NNEG = 5
VOCAB = 1000000
W = 2 * D
BPW = B // NW
CH = 64
NCHUNK = BPW // CH
GRP = CH // L
DV = D // L
NSC = 1 + NNEG

VB = 16384
HB = VB // 2
PG = -(-(VOCAB + 1) // VB)
NPACK = PG * HB


def _pack_body(a_ref, o_ref):
    x = a_ref[...]
    o_ref[...] = jnp.concatenate([x[:, :HB].T, x[:, HB:].T], axis=1)


@jax.jit
def _pack(table_t):
    return pl.pallas_call(
        _pack_body,
        grid=(PG,),
        in_specs=[pl.BlockSpec((D, VB), lambda i: (0, i))],
        out_specs=pl.BlockSpec((HB, W), lambda i: (i, 0)),
        out_shape=jax.ShapeDtypeStruct((NPACK, W), jnp.float32),
    )(table_t)


@jax.jit
def _pack_tail(table_t, packed):
    return pl.pallas_call(
        lambda a_ref, alias_ref, o_ref: _pack_body(a_ref, o_ref),
        grid=(1,),
        in_specs=[
            pl.BlockSpec((D, VB), lambda i: (0, PG - 1)),
            pl.BlockSpec(memory_space=pl.ANY),
        ],
        out_specs=pl.BlockSpec((HB, W), lambda i: (PG - 1, 0)),
        out_shape=jax.ShapeDtypeStruct((NPACK, W), jnp.float32),
        input_output_aliases={1: 0},
    )(table_t, packed)


NSB = VOCAB // VB
NPAIR = NSB * (HB // 128)
PPW = NPAIR // NW


def _sc_pack_body(vt, out, ina, inb, outb, sems):
    wid = lax.axis_index("s") * NC + lax.axis_index("c")
    lane = lax.iota(jnp.int32, L)
    t0 = wid * PPW

    def load(t, slot):
        col = (t // 64) * VB + (t % 64) * 128
        pltpu.async_copy(vt.at[:, pl.ds(col, 128)], ina.at[slot], sems.at[slot])
        pltpu.async_copy(vt.at[:, pl.ds(col + HB, 128)], inb.at[slot],
                         sems.at[slot])

    def drain_in(slot):
        pltpu.make_async_copy(
            vt.at[:, pl.ds(0, 128)], ina.at[slot], sems.at[slot]).wait()
        pltpu.make_async_copy(
            vt.at[:, pl.ds(0, 128)], inb.at[slot], sems.at[slot]).wait()

    def drain_out(slot):
        pltpu.make_async_copy(
            outb.at[slot], out.at[pl.ds(0, 128), :], sems.at[2 + slot]).wait()

    load(t0, 0)

    def super_body(i2, _):
        for b in range(2):
            i = i2 * 2 + b
            t = t0 + i
            slot, nslot = b, 1 - b

            @pl.when(i + 1 < PPW)
            def _():
                load(t + 1, nslot)

            drain_in(slot)

            @pl.when(i >= 2)
            def _():
                drain_out(slot)

            def row_body(vv, _):
                vvs = jnp.full((L,), vv, jnp.int32)
                for k in range(8):
                    src = ina if k < 4 else inb
                    g = plsc.load_gather(src.at[slot],
                                         [(k % 4) * L + lane, vvs])
                    outb[slot, vv, pl.ds(k * L, L)] = g
                return 0

            lax.fori_loop(0, 128, row_body, 0)
            pltpu.async_copy(
                outb.at[slot],
                out.at[pl.ds((t // 64) * HB + (t % 64) * 128, 128), :],
                sems.at[2 + slot])
        return 0

    lax.fori_loop(0, PPW // 2, super_body, 0)
    drain_out(0)
    drain_out(1)


@jax.jit
def _sc_pack_v(vt):
    mesh = plsc.VectorSubcoreMesh(core_axis_name="c", subcore_axis_name="s")
    return pl.kernel(
        _sc_pack_body,
        out_type=jax.ShapeDtypeStruct((NPACK, W), jnp.float32),
        mesh=mesh,
        compiler_params=pltpu.CompilerParams(
            needs_layout_passes=False, use_tc_tiling_on_sc=True),
        scratch_types=[
            pltpu.VMEM((2, D, 128), jnp.float32),
            pltpu.VMEM((2, D, 128), jnp.float32),
            pltpu.VMEM((2, 128, W), jnp.float32),
            pltpu.SemaphoreType.DMA((4,)),
        ],
    )(vt)


def _splat(ref, idx):
    return plsc.load_gather(ref, [jnp.full((L,), idx, jnp.int32)])


def _sc_body(uraw, vraw, nraw, up, vp, out,
             uraw_v, vraw_v, nraw_v, upidx, uoff_v, vpidx, voff_v,
             npidx, noff_v, urows_v, vrows_v, nrows_v, scores_v, sem):
    wid = lax.axis_index("s") * NC + lax.axis_index("c")
    base = wid * BPW
    pltpu.sync_copy(uraw.at[pl.ds(base, BPW)], uraw_v)
    pltpu.sync_copy(vraw.at[pl.ds(base, BPW)], vraw_v)
    pltpu.sync_copy(nraw.at[pl.ds(base * NNEG, BPW * NNEG)], nraw_v)

    lane = lax.iota(jnp.int32, L)

    def idx_split(raw_ref, pidx_ref, off_ref, src_off, n):
        for i in range(n // L):
            v = raw_ref[pl.ds(src_off + i * L, L)]
            pidx_ref[pl.ds(i * L, L)] = jnp.bitwise_or(
                lax.shift_left(lax.shift_right_logical(v, 14), 13),
                jnp.bitwise_and(v, HB - 1))
            off_ref[pl.ds(i * L, L)] = lax.shift_left(
                jnp.bitwise_and(lax.shift_right_logical(v, 13), 1), 6)

    def chunk_body(c, _):
        idx_split(uraw_v, upidx, uoff_v, c * CH, CH)
        idx_split(vraw_v, vpidx, voff_v, c * CH, CH)
        idx_split(nraw_v, npidx, noff_v, c * (CH * NNEG), CH * NNEG)
        cps = [
            pltpu.async_copy(up.at[upidx], urows_v, sem),
            pltpu.async_copy(vp.at[vpidx], vrows_v, sem),
        ]
        for q in range(NNEG):
            cps.append(pltpu.async_copy(
                vp.at[npidx.at[pl.ds(q * CH, CH)]],
                nrows_v.at[pl.ds(q * CH, CH)], sem))
        for cp in cps:
            cp.wait()

        def group_body(g, _):
            accs = [jnp.zeros((L,), jnp.float32) for _ in range(NSC)]
            for j in range(L):
                e = g * L + j
                uo = _splat(uoff_v, e)
                vo = _splat(voff_v, e)
                us = [plsc.load_gather(urows_v, [jnp.full((L,), e, jnp.int32),
                                                 uo + (k * L + lane)])
                      for k in range(DV)]
                vs = [plsc.load_gather(vrows_v, [jnp.full((L,), e, jnp.int32),
                                                 vo + (k * L + lane)])
                      for k in range(DV)]
                s = jnp.sum(sum(u * v for u, v in zip(us, vs)))
                accs[0] = jnp.where(lane == j, s, accs[0])
                for q in range(NNEG):
                    p = e * NNEG + q
                    no = _splat(noff_v, p)
                    ns = [plsc.load_gather(nrows_v,
                                           [jnp.full((L,), p, jnp.int32),
                                            no + (k * L + lane)])
                          for k in range(DV)]
                    s = jnp.sum(sum(u * n for u, n in zip(us, ns)))
                    accs[1 + q] = jnp.where(lane == j, s, accs[1 + q])
            for r in range(NSC):
                scores_v[pl.ds(r * BPW + c * CH + g * L, L)] = accs[r]
            return 0

        lax.fori_loop(0, GRP, group_body, 0)
        return 0

    lax.fori_loop(0, NCHUNK, chunk_body, 0)
    pltpu.sync_copy(scores_v, out.at[pl.ds(base * NSC, BPW * NSC)])


@jax.jit
def _sc_scores(uraw, vraw, nraw, up, vp):
    mesh = plsc.VectorSubcoreMesh(core_axis_name="c", subcore_axis_name="s")
    return pl.kernel(
        _sc_body,
        out_type=jax.ShapeDtypeStruct((B * NSC,), jnp.float32),
        mesh=mesh,
        compiler_params=pltpu.CompilerParams(
            needs_layout_passes=False, use_tc_tiling_on_sc=True),
        scratch_types=[
            pltpu.VMEM((BPW,), jnp.int32),
            pltpu.VMEM((BPW,), jnp.int32),
            pltpu.VMEM((BPW * NNEG,), jnp.int32),
            pltpu.VMEM((CH,), jnp.int32),
            pltpu.VMEM((CH,), jnp.int32),
            pltpu.VMEM((CH,), jnp.int32),
            pltpu.VMEM((CH,), jnp.int32),
            pltpu.VMEM((CH * NNEG,), jnp.int32),
            pltpu.VMEM((CH * NNEG,), jnp.int32),
            pltpu.VMEM((CH, W), jnp.float32),
            pltpu.VMEM((CH, W), jnp.float32),
            pltpu.VMEM((CH * NNEG, W), jnp.float32),
            pltpu.VMEM((BPW * NSC,), jnp.float32),
            pltpu.SemaphoreType.DMA,
        ],
    )(uraw, vraw, nraw, up, vp)


def _loss_body(scores_ref, out_ref):
    s = scores_ref[...]
    r = lax.broadcasted_iota(jnp.int32, s.shape, 1)
    x = jnp.where(r == 0, s, -s)
    ls = jax.nn.log_sigmoid(x)
    out_ref[...] = jnp.full((1, 1), -jnp.sum(ls) / B, jnp.float32)


@jax.jit
def _loss(scores):
    out = pl.pallas_call(
        _loss_body,
        out_shape=jax.ShapeDtypeStruct((1, 1), jnp.float32),
    )(scores.reshape(NW, NSC, BPW))
    return out[0, 0]


def kernel(pos_u, pos_v, neg_v, u_weight, v_weight):
    vp0 = _sc_pack_v(v_weight.T)
    up = _pack(u_weight.T)
    vp = _pack_tail(v_weight.T, vp0)
    scores = _sc_scores(pos_u, pos_v, neg_v.reshape(-1), up, vp)
    return _loss(scores)


# speedup vs baseline: 2.9600x; 2.9600x over previous
import functools

import jax
import jax.numpy as jnp
from jax import lax
from jax.experimental import pallas as pl
from jax.experimental.pallas import tpu as pltpu
from jax.experimental.pallas import tpu_sc as plsc

NC = 2
NSUB = 16
NW = NC * NSUB
L = 16

B = 16384
D = 64
NNEG = 5
VOCAB = 1000000
W = 2 * D
BPW = B // NW
CH = 64
NCHUNK = BPW // CH
GRP = CH // L
DV = D // L
NSC = 1 + NNEG

VB = 16384
HB = VB // 2
PG = -(-(VOCAB + 1) // VB)
NPACK = PG * HB


def _pack_body(a_ref, o_ref):
    x = a_ref[...]
    o_ref[...] = jnp.concatenate([x[:, :HB].T, x[:, HB:].T], axis=1)


@jax.jit
def _pack(table_t):
    return pl.pallas_call(
        _pack_body,
        grid=(PG,),
        in_specs=[pl.BlockSpec((D, VB), lambda i: (0, i))],
        out_specs=pl.BlockSpec((HB, W), lambda i: (i, 0)),
        out_shape=jax.ShapeDtypeStruct((NPACK, W), jnp.float32),
    )(table_t)


def _splat(ref, idx):
    return plsc.load_gather(ref, [jnp.full((L,), idx, jnp.int32)])


def _sc_body(uraw, vraw, nraw, up, vp, out,
             uraw_v, vraw_v, nraw_v, upidx, uoff_v, vpidx, voff_v,
             npidx, noff_v, urows_v, vrows_v, nrows_v, scores_v, sem):
    wid = lax.axis_index("s") * NC + lax.axis_index("c")
    base = wid * BPW
    pltpu.sync_copy(uraw.at[pl.ds(base, BPW)], uraw_v)
    pltpu.sync_copy(vraw.at[pl.ds(base, BPW)], vraw_v)
    pltpu.sync_copy(nraw.at[pl.ds(base * NNEG, BPW * NNEG)], nraw_v)

    lane = lax.iota(jnp.int32, L)

    def idx_split(raw_ref, pidx_ref, off_ref, src_off, n):
        for i in range(n // L):
            v = raw_ref[pl.ds(src_off + i * L, L)]
            pidx_ref[pl.ds(i * L, L)] = jnp.bitwise_or(
                lax.shift_left(lax.shift_right_logical(v, 14), 13),
                jnp.bitwise_and(v, HB - 1))
            off_ref[pl.ds(i * L, L)] = lax.shift_left(
                jnp.bitwise_and(lax.shift_right_logical(v, 13), 1), 6)

    def chunk_body(c, _):
        idx_split(uraw_v, upidx, uoff_v, c * CH, CH)
        idx_split(vraw_v, vpidx, voff_v, c * CH, CH)
        idx_split(nraw_v, npidx, noff_v, c * (CH * NNEG), CH * NNEG)
        cps = [
            pltpu.async_copy(up.at[upidx], urows_v, sem),
            pltpu.async_copy(vp.at[vpidx], vrows_v, sem),
        ]
        for q in range(NNEG):
            cps.append(pltpu.async_copy(
                vp.at[npidx.at[pl.ds(q * CH, CH)]],
                nrows_v.at[pl.ds(q * CH, CH)], sem))
        for cp in cps:
            cp.wait()

        def group_body(g, _):
            accs = [jnp.zeros((L,), jnp.float32) for _ in range(NSC)]
            for j in range(L):
                e = g * L + j
                uo = _splat(uoff_v, e)
                vo = _splat(voff_v, e)
                us = [plsc.load_gather(urows_v, [jnp.full((L,), e, jnp.int32),
                                                 uo + (k * L + lane)])
                      for k in range(DV)]
                vs = [plsc.load_gather(vrows_v, [jnp.full((L,), e, jnp.int32),
                                                 vo + (k * L + lane)])
                      for k in range(DV)]
                s = jnp.sum(sum(u * v for u, v in zip(us, vs)))
                accs[0] = jnp.where(lane == j, s, accs[0])
                for q in range(NNEG):
                    p = e * NNEG + q
                    no = _splat(noff_v, p)
                    ns = [plsc.load_gather(nrows_v,
                                           [jnp.full((L,), p, jnp.int32),
                                            no + (k * L + lane)])
                          for k in range(DV)]
                    s = jnp.sum(sum(u * n for u, n in zip(us, ns)))
                    accs[1 + q] = jnp.where(lane == j, s, accs[1 + q])
            for r in range(NSC):
                scores_v[pl.ds(r * BPW + c * CH + g * L, L)] = accs[r]
            return 0

        lax.fori_loop(0, GRP, group_body, 0)
        return 0

    lax.fori_loop(0, NCHUNK, chunk_body, 0)
    pltpu.sync_copy(scores_v, out.at[pl.ds(base * NSC, BPW * NSC)])


@jax.jit
def _sc_scores(uraw, vraw, nraw, up, vp):
    mesh = plsc.VectorSubcoreMesh(core_axis_name="c", subcore_axis_name="s")
    return pl.kernel(
        _sc_body,
        out_type=jax.ShapeDtypeStruct((B * NSC,), jnp.float32),
        mesh=mesh,
        compiler_params=pltpu.CompilerParams(
            needs_layout_passes=False, use_tc_tiling_on_sc=True),
        scratch_types=[
            pltpu.VMEM((BPW,), jnp.int32),
            pltpu.VMEM((BPW,), jnp.int32),
            pltpu.VMEM((BPW * NNEG,), jnp.int32),
            pltpu.VMEM((CH,), jnp.int32),
            pltpu.VMEM((CH,), jnp.int32),
            pltpu.VMEM((CH,), jnp.int32),
            pltpu.VMEM((CH,), jnp.int32),
            pltpu.VMEM((CH * NNEG,), jnp.int32),
            pltpu.VMEM((CH * NNEG,), jnp.int32),
            pltpu.VMEM((CH, W), jnp.float32),
            pltpu.VMEM((CH, W), jnp.float32),
            pltpu.VMEM((CH * NNEG, W), jnp.float32),
            pltpu.VMEM((BPW * NSC,), jnp.float32),
            pltpu.SemaphoreType.DMA,
        ],
    )(uraw, vraw, nraw, up, vp)


def _loss_body(scores_ref, out_ref):
    s = scores_ref[...]
    r = lax.broadcasted_iota(jnp.int32, s.shape, 1)
    x = jnp.where(r == 0, s, -s)
    ls = jax.nn.log_sigmoid(x)
    out_ref[...] = jnp.full((1, 1), -jnp.sum(ls) / B, jnp.float32)


@jax.jit
def _loss(scores):
    out = pl.pallas_call(
        _loss_body,
        out_shape=jax.ShapeDtypeStruct((1, 1), jnp.float32),
    )(scores.reshape(NW, NSC, BPW))
    return out[0, 0]


def kernel(pos_u, pos_v, neg_v, u_weight, v_weight):
    up = _pack(u_weight.T)
    vp = _pack(v_weight.T)
    scores = _sc_scores(pos_u, pos_v, neg_v.reshape(-1), up, vp)
    return _loss(scores)


# speedup vs baseline: 2.9601x; 1.0000x over previous
import functools

import jax
import jax.numpy as jnp
from jax import lax
from jax.experimental import pallas as pl
from jax.experimental.pallas import tpu as pltpu
from jax.experimental.pallas import tpu_sc as plsc

NC = 2
NSUB = 16
NW = NC * NSUB
L = 16

B = 16384
D = 64
NNEG = 5
VOCAB = 1000000
W = 2 * D
BPW = B // NW
CH = 32
NCHUNK = BPW // CH
GRP = CH // L
DV = D // L
NSC = 1 + NNEG

VB = 16384
HB = VB // 2
PG = -(-(VOCAB + 1) // VB)
NPACK = PG * HB


def _pack_body(a_ref, o_ref):
    x = a_ref[...]
    o_ref[...] = jnp.concatenate([x[:, :HB].T, x[:, HB:].T], axis=1)


@jax.jit
def _pack(table_t):
    return pl.pallas_call(
        _pack_body,
        grid=(PG,),
        in_specs=[pl.BlockSpec((D, VB), lambda i: (0, i))],
        out_specs=pl.BlockSpec((HB, W), lambda i: (i, 0)),
        out_shape=jax.ShapeDtypeStruct((NPACK, W), jnp.float32),
    )(table_t)


def _splat(ref, idx):
    return plsc.load_gather(ref, [jnp.full((L,), idx, jnp.int32)])


def _sc_body(uraw, vraw, nraw, up, vp, out,
             uraw_v, vraw_v, nraw_v, upidx, uoff_v, vpidx, voff_v,
             npidx, noff_v, urows_v, vrows_v, nrows_v, scores_v, sem):
    wid = lax.axis_index("s") * NC + lax.axis_index("c")
    base = wid * BPW
    pltpu.sync_copy(uraw.at[pl.ds(base, BPW)], uraw_v)
    pltpu.sync_copy(vraw.at[pl.ds(base, BPW)], vraw_v)
    pltpu.sync_copy(nraw.at[pl.ds(base * NNEG, BPW * NNEG)], nraw_v)

    lane = lax.iota(jnp.int32, L)

    def idx_split(raw_ref, pidx_ref, off_ref, n):
        for i in range(n // L):
            v = raw_ref[pl.ds(i * L, L)]
            pidx_ref[pl.ds(i * L, L)] = jnp.bitwise_or(
                lax.shift_left(lax.shift_right_logical(v, 14), 13),
                jnp.bitwise_and(v, HB - 1))
            off_ref[pl.ds(i * L, L)] = lax.shift_left(
                jnp.bitwise_and(lax.shift_right_logical(v, 13), 1), 6)

    idx_split(uraw_v, upidx, uoff_v, BPW)
    idx_split(vraw_v, vpidx, voff_v, BPW)
    idx_split(nraw_v, npidx, noff_v, BPW * NNEG)

    def fire(c, slot):
        pltpu.async_copy(up.at[upidx.at[pl.ds(c * CH, CH)]],
                         urows_v.at[slot], sem)
        pltpu.async_copy(vp.at[vpidx.at[pl.ds(c * CH, CH)]],
                         vrows_v.at[slot], sem)
        for q in range(NNEG):
            pltpu.async_copy(
                vp.at[npidx.at[pl.ds(c * (CH * NNEG) + q * CH, CH)]],
                nrows_v.at[slot].at[pl.ds(q * CH, CH)], sem)

    def drain(slot):
        pltpu.make_async_copy(up.at[pl.ds(0, CH)], urows_v.at[slot],
                              sem).wait()
        pltpu.make_async_copy(vp.at[pl.ds(0, CH)], vrows_v.at[slot],
                              sem).wait()
        for q in range(NNEG):
            pltpu.make_async_copy(
                vp.at[pl.ds(0, CH)], nrows_v.at[slot].at[pl.ds(q * CH, CH)],
                sem).wait()

    fire(0, 0)

    def super_body(c2, _):
        for b in range(2):
            c = c2 * 2 + b
            slot = b

            @pl.when(c + 1 < NCHUNK)
            def _():
                fire(c + 1, 1 - b)

            drain(slot)

            def group_body(g, _):
                accs = [jnp.zeros((L,), jnp.float32) for _ in range(NSC)]
                for j in range(L):
                    e = g * L + j
                    uo = _splat(uoff_v, c * CH + e)
                    vo = _splat(voff_v, c * CH + e)
                    ej = jnp.full((L,), e, jnp.int32)
                    us = [plsc.load_gather(urows_v.at[slot],
                                           [ej, uo + (k * L + lane)])
                          for k in range(DV)]
                    vs = [plsc.load_gather(vrows_v.at[slot],
                                           [ej, vo + (k * L + lane)])
                          for k in range(DV)]
                    s = jnp.sum(sum(u * v for u, v in zip(us, vs)))
                    accs[0] = jnp.where(lane == j, s, accs[0])
                    for q in range(NNEG):
                        p = e * NNEG + q
                        no = _splat(noff_v, c * (CH * NNEG) + p)
                        ns = [plsc.load_gather(
                                  nrows_v.at[slot],
                                  [jnp.full((L,), p, jnp.int32),
                                   no + (k * L + lane)])
                              for k in range(DV)]
                        s = jnp.sum(sum(u * n for u, n in zip(us, ns)))
                        accs[1 + q] = jnp.where(lane == j, s, accs[1 + q])
                for r in range(NSC):
                    scores_v[pl.ds(r * BPW + c * CH + g * L, L)] = accs[r]
                return 0

            lax.fori_loop(0, GRP, group_body, 0)
        return 0

    lax.fori_loop(0, NCHUNK // 2, super_body, 0)
    pltpu.sync_copy(scores_v, out.at[pl.ds(base * NSC, BPW * NSC)])


@jax.jit
def _sc_scores(uraw, vraw, nraw, up, vp):
    mesh = plsc.VectorSubcoreMesh(core_axis_name="c", subcore_axis_name="s")
    return pl.kernel(
        _sc_body,
        out_type=jax.ShapeDtypeStruct((B * NSC,), jnp.float32),
        mesh=mesh,
        compiler_params=pltpu.CompilerParams(
            needs_layout_passes=False, use_tc_tiling_on_sc=True),
        scratch_types=[
            pltpu.VMEM((BPW,), jnp.int32),
            pltpu.VMEM((BPW,), jnp.int32),
            pltpu.VMEM((BPW * NNEG,), jnp.int32),
            pltpu.VMEM((BPW,), jnp.int32),
            pltpu.VMEM((BPW,), jnp.int32),
            pltpu.VMEM((BPW,), jnp.int32),
            pltpu.VMEM((BPW,), jnp.int32),
            pltpu.VMEM((BPW * NNEG,), jnp.int32),
            pltpu.VMEM((BPW * NNEG,), jnp.int32),
            pltpu.VMEM((2, CH, W), jnp.float32),
            pltpu.VMEM((2, CH, W), jnp.float32),
            pltpu.VMEM((2, CH * NNEG, W), jnp.float32),
            pltpu.VMEM((BPW * NSC,), jnp.float32),
            pltpu.SemaphoreType.DMA,
        ],
    )(uraw, vraw, nraw, up, vp)


def _loss_body(scores_ref, out_ref):
    s = scores_ref[...]
    r = lax.broadcasted_iota(jnp.int32, s.shape, 1)
    x = jnp.where(r == 0, s, -s)
    ls = jax.nn.log_sigmoid(x)
    out_ref[...] = jnp.full((1, 1), -jnp.sum(ls) / B, jnp.float32)


@jax.jit
def _loss(scores):
    out = pl.pallas_call(
        _loss_body,
        out_shape=jax.ShapeDtypeStruct((1, 1), jnp.float32),
    )(scores.reshape(NW, NSC, BPW))
    return out[0, 0]


def kernel(pos_u, pos_v, neg_v, u_weight, v_weight):
    up = _pack(u_weight.T)
    vp = _pack(v_weight.T)
    scores = _sc_scores(pos_u, pos_v, neg_v.reshape(-1), up, vp)
    return _loss(scores)
